# baseline (speedup 1.0000x reference)
import functools

import jax
import jax.numpy as jnp
from jax.experimental import pallas as pl
from jax.experimental.pallas import tpu as pltpu


def _fused_body(inv_hw, x_ref, wc_ref, bc_ref, wp_ref, bp_ref,
                scores_ref, deltas_ref):
    pooled = jnp.sum(x_ref[...], axis=0) * inv_hw
    dn = (((1,), (0,)), ((), ()))
    scores_ref[...] = jax.lax.dot_general(
        pooled, wc_ref[...], dn,
        preferred_element_type=jnp.float32) + bc_ref[...][None, :]
    deltas_ref[...] = jax.lax.dot_general(
        pooled, wp_ref[...], dn,
        preferred_element_type=jnp.float32) + bp_ref[...][None, :]


def kernel(x, w_cls, b_cls, w_pred, b_pred):
    num_classes = w_cls.shape[0]
    nc4 = w_pred.shape[0]

    if x.ndim == 4:
        N, C, H, W = x.shape
        hw = H * W
        xt = x.transpose(2, 3, 0, 1).reshape(hw, N, C)
    else:
        N, C = x.shape
        hw = 1
        xt = x.reshape(1, N, C)


    if N % 128 == 0:
        tn = 128
    elif N % 8 == 0:
        tn = 8
    else:
        tn = N
    grid = (N // tn,)

    itemsize = jnp.dtype(x.dtype).itemsize
    cost = pl.CostEstimate(
        flops=int(N * C * hw + 2 * N * C * (num_classes + nc4)),
        transcendentals=0,
        bytes_accessed=int(N * C * hw * itemsize
                           + (w_cls.size + w_pred.size) * 4
                           + N * (num_classes + nc4) * 4),
    )

    scores, deltas = pl.pallas_call(
        functools.partial(_fused_body, 1.0 / float(hw)),
        out_shape=(jax.ShapeDtypeStruct((N, num_classes), jnp.float32),
                   jax.ShapeDtypeStruct((N, nc4), jnp.float32)),
        grid=grid,
        in_specs=[
            pl.BlockSpec((hw, tn, C), lambda i: (0, i, 0)),
            pl.BlockSpec((C, num_classes), lambda i: (0, 0)),
            pl.BlockSpec((num_classes,), lambda i: (0,)),
            pl.BlockSpec((C, nc4), lambda i: (0, 0)),
            pl.BlockSpec((nc4,), lambda i: (0,)),
        ],
        out_specs=[
            pl.BlockSpec((tn, num_classes), lambda i: (i, 0)),
            pl.BlockSpec((tn, nc4), lambda i: (i, 0)),
        ],
        compiler_params=pltpu.CompilerParams(
            dimension_semantics=("parallel",),
            vmem_limit_bytes=48 * 1024 * 1024,
        ),
        cost_estimate=cost,
    )(xt, w_cls.T, b_cls, w_pred.T, b_pred)
    return scores, deltas


# speedup vs baseline: 1.0558x; 1.0558x over previous
import functools

import jax
import jax.numpy as jnp
from jax.experimental import pallas as pl
from jax.experimental.pallas import tpu as pltpu


def _fused_body(inv_hw, x_ref, wc_hbm, bc_hbm, wp_hbm, bp_hbm,
                scores_ref, deltas_ref, wc_v, bc_v, wp_v, bp_v, sems):
    cps = [
        pltpu.make_async_copy(wc_hbm, wc_v, sems.at[0]),
        pltpu.make_async_copy(bc_hbm, bc_v, sems.at[1]),
        pltpu.make_async_copy(wp_hbm, wp_v, sems.at[2]),
        pltpu.make_async_copy(bp_hbm, bp_v, sems.at[3]),
    ]
    for cp in cps:
        cp.start()
    pooled = jnp.sum(x_ref[...], axis=0) * inv_hw
    for cp in cps:
        cp.wait()
    dn = (((1,), (1,)), ((), ()))
    scores_ref[...] = jax.lax.dot_general(
        wc_v[...], pooled, dn,
        preferred_element_type=jnp.float32) + bc_v[...][:, None]
    deltas_ref[...] = jax.lax.dot_general(
        wp_v[...], pooled, dn,
        preferred_element_type=jnp.float32) + bp_v[...][:, None]


def kernel(x, w_cls, b_cls, w_pred, b_pred):
    num_classes = w_cls.shape[0]
    nc4 = w_pred.shape[0]

    if x.ndim == 4:
        N, C, H, W = x.shape
        hw = H * W
        xt = x.transpose(2, 3, 0, 1).reshape(hw, N, C)
    else:
        N, C = x.shape
        hw = 1
        xt = x.reshape(1, N, C)

    if N % 128 == 0:
        tn = 128
    elif N % 8 == 0:
        tn = 8
    else:
        tn = N
    grid = (N // tn,)

    itemsize = jnp.dtype(x.dtype).itemsize
    cost = pl.CostEstimate(
        flops=int(N * C * hw + 2 * N * C * (num_classes + nc4)),
        transcendentals=0,
        bytes_accessed=int(N * C * hw * itemsize
                           + (w_cls.size + w_pred.size) * 4
                           + N * (num_classes + nc4) * 4),
    )

    scores_t, deltas_t = pl.pallas_call(
        functools.partial(_fused_body, 1.0 / float(hw)),
        out_shape=(jax.ShapeDtypeStruct((num_classes, N), jnp.float32),
                   jax.ShapeDtypeStruct((nc4, N), jnp.float32)),
        grid=grid,
        in_specs=[
            pl.BlockSpec((hw, tn, C), lambda i: (0, i, 0)),
            pl.BlockSpec(memory_space=pl.ANY),
            pl.BlockSpec(memory_space=pl.ANY),
            pl.BlockSpec(memory_space=pl.ANY),
            pl.BlockSpec(memory_space=pl.ANY),
        ],
        out_specs=[
            pl.BlockSpec((num_classes, tn), lambda i: (0, i)),
            pl.BlockSpec((nc4, tn), lambda i: (0, i)),
        ],
        scratch_shapes=[
            pltpu.VMEM((num_classes, C), jnp.float32),
            pltpu.VMEM((num_classes,), jnp.float32),
            pltpu.VMEM((nc4, C), jnp.float32),
            pltpu.VMEM((nc4,), jnp.float32),
            pltpu.SemaphoreType.DMA((4,)),
        ],
        compiler_params=pltpu.CompilerParams(
            dimension_semantics=("parallel",),
            vmem_limit_bytes=48 * 1024 * 1024,
        ),
        cost_estimate=cost,
    )(xt, w_cls, b_cls, w_pred, b_pred)
    return scores_t.T, deltas_t.T
